# baseline (speedup 1.0000x reference)
import functools

import jax
import jax.numpy as jnp
from jax import lax
from jax.experimental import pallas as pl
from jax.experimental.pallas import tpu as pltpu
from jax.experimental.pallas import tpu_sc as plsc

_A = 12.5
_RH = 0.65
_CA = 420.0
_COEF = _A * _RH / _CA

_NUM_FGS = 1
_LANES = 16


def _sc_body(fgs_hbm, gs0_hbm, a1_hbm, out_hbm, idx_v, g_v, a_v, out_v, sem):
    nc = plsc.get_sparse_core_info().num_cores
    wid = lax.axis_index("s") * nc + lax.axis_index("c")

    @pl.when(wid == 0)
    def _():
        pltpu.sync_copy(fgs_hbm, idx_v)
        pltpu.async_copy(gs0_hbm.at[idx_v], g_v.at[pl.ds(0, _NUM_FGS)], sem).wait()
        pltpu.async_copy(a1_hbm.at[idx_v], a_v.at[pl.ds(0, _NUM_FGS)], sem).wait()
        out_v[...] = g_v[...] + a_v[...] * _COEF
        pltpu.sync_copy(out_v.at[pl.ds(0, _NUM_FGS)], out_hbm)


def kernel(gs0, a1, FGs):
    fgs = FGs.astype(jnp.int32)
    mesh = plsc.VectorSubcoreMesh(core_axis_name="c", subcore_axis_name="s")
    run = functools.partial(
        pl.kernel,
        mesh=mesh,
        out_type=jax.ShapeDtypeStruct((_NUM_FGS,), jnp.float32),
        scratch_types=[
            pltpu.VMEM((_NUM_FGS,), jnp.int32),
            pltpu.VMEM((_LANES,), jnp.float32),
            pltpu.VMEM((_LANES,), jnp.float32),
            pltpu.VMEM((_LANES,), jnp.float32),
            pltpu.SemaphoreType.DMA,
        ],
    )(_sc_body)
    return run(fgs, gs0, a1)
